# baseline (speedup 1.0000x reference)
import functools

import jax
import jax.numpy as jnp
from jax import lax
from jax.experimental import pallas as pl
from jax.experimental.pallas import tpu as pltpu
from jax.experimental.pallas import tpu_sc as plsc

BATCH = 64
SEQ = 2048
D_IN = 1024
D_PROJ = 256

_NUM_W = 4
_ROWS_PER_W = 32


def _gather_body(start_hbm, end_hbm, table_hbm, out_hbm, ids_v, idx_v, rows_v, sem):
    wid = lax.axis_index("s")
    b0 = (wid & 1) * _ROWS_PER_W

    @pl.when(wid < 2)
    def _():
        pltpu.sync_copy(start_hbm.at[pl.ds(b0, _ROWS_PER_W)], ids_v)

    @pl.when(wid >= 2)
    def _():
        pltpu.sync_copy(end_hbm.at[pl.ds(b0, _ROWS_PER_W)], ids_v)

    for j in (0, 16):
        idx_v[pl.ds(j, 16)] = (
            ids_v[pl.ds(j, 16)] + (b0 + j + lax.iota(jnp.int32, 16)) * SEQ
        )

    pltpu.async_copy(table_hbm.at[idx_v], rows_v, sem).wait()
    pltpu.sync_copy(rows_v, out_hbm.at[pl.ds(wid * _ROWS_PER_W, _ROWS_PER_W)])


_gather_rows = functools.partial(
    pl.kernel,
    mesh=plsc.VectorSubcoreMesh(core_axis_name="c", subcore_axis_name="s",
                                num_cores=1, num_subcores=_NUM_W),
    out_type=jax.ShapeDtypeStruct((2 * BATCH, D_IN), jnp.float32),
    scratch_types=[
        pltpu.VMEM((_ROWS_PER_W,), jnp.int32),
        pltpu.VMEM((_ROWS_PER_W,), jnp.int32),
        pltpu.VMEM((_ROWS_PER_W, D_IN), jnp.float32),
        pltpu.SemaphoreType.DMA,
    ],
)(_gather_body)


def _proj_body(g_ref, w_ref, b_ref, o_ref):
    r = lax.dot_general(
        g_ref[...], w_ref[...],
        dimension_numbers=(((1,), (1,)), ((), ())),
        preferred_element_type=jnp.float32,
    )
    r = r + b_ref[...]
    o_ref[:, :D_PROJ] = r[:BATCH, :]
    o_ref[:, D_PROJ:] = r[BATCH:, :]


def kernel(encoded_input, start_ids, end_ids, W, b):
    table = encoded_input.reshape(BATCH * SEQ, D_IN)
    gathered = _gather_rows(
        start_ids.astype(jnp.int32), end_ids.astype(jnp.int32), table
    )
    return pl.pallas_call(
        _proj_body,
        out_shape=jax.ShapeDtypeStruct((BATCH, 2 * D_PROJ), jnp.float32),
    )(gathered, W, b.reshape(1, D_PROJ))


# speedup vs baseline: 1.0995x; 1.0995x over previous
import functools

import jax
import jax.numpy as jnp
from jax import lax
from jax.experimental import pallas as pl
from jax.experimental.pallas import tpu as pltpu
from jax.experimental.pallas import tpu_sc as plsc

BATCH = 64
SEQ = 2048
D_IN = 1024
D_PROJ = 256

_NUM_W = 16
_ROWS_PER_W = 4


def _gather_body(idx_hbm, table_hbm, out_hbm, idx_v, rows_v, sem):
    wid = lax.axis_index("s")

    pltpu.sync_copy(idx_hbm.at[wid], idx_v)
    pltpu.async_copy(table_hbm.at[idx_v], rows_v, sem).wait()
    pltpu.sync_copy(rows_v, out_hbm.at[pl.ds(wid * _ROWS_PER_W, _ROWS_PER_W)])


_gather_ends = functools.partial(
    pl.kernel,
    mesh=plsc.VectorSubcoreMesh(core_axis_name="c", subcore_axis_name="s",
                                num_cores=1),
    out_type=jax.ShapeDtypeStruct((BATCH, D_IN), jnp.float32),
    scratch_types=[
        pltpu.VMEM((_ROWS_PER_W,), jnp.int32),
        pltpu.VMEM((_ROWS_PER_W, D_IN), jnp.float32),
        pltpu.SemaphoreType.DMA,
    ],
)(_gather_body)


def _matmul(g):
    def f(w_ref):
        return lax.dot_general(
            g, w_ref[...],
            dimension_numbers=(((1,), (1,)), ((), ())),
            preferred_element_type=jnp.float32,
        )
    return f


def _start_body(idx_ref, table_ref, w_ref, b_ref, o_ref, g_ref, sem):
    cps = []
    for i in range(BATCH):
        cp = pltpu.make_async_copy(
            table_ref.at[pl.ds(idx_ref[i], 1)], g_ref.at[pl.ds(i, 1)], sem)
        cp.start()
        cps.append(cp)
    for cp in cps:
        cp.wait()
    o_ref[...] = _matmul(g_ref[...])(w_ref) + b_ref[...]


def _final_body(g_ref, w_ref, b_ref, start_ref, o_ref):
    o_ref[:, :D_PROJ] = start_ref[...]
    o_ref[:, D_PROJ:] = _matmul(g_ref[...])(w_ref) + b_ref[...]


def kernel(encoded_input, start_ids, end_ids, W, b):
    table = encoded_input.reshape(BATCH * SEQ, D_IN)
    offs = jnp.arange(BATCH, dtype=jnp.int32) * SEQ
    idx_start = start_ids.astype(jnp.int32) + offs
    idx_end = (end_ids.astype(jnp.int32) + offs).reshape(_NUM_W, _ROWS_PER_W)
    b2 = b.reshape(1, D_PROJ)

    gathered_ends = _gather_ends(idx_end, table)

    start_half = pl.pallas_call(
        _start_body,
        in_specs=[
            pl.BlockSpec(memory_space=pltpu.SMEM),
            pl.BlockSpec(memory_space=pltpu.MemorySpace.HBM),
            pl.BlockSpec(memory_space=pltpu.VMEM),
            pl.BlockSpec(memory_space=pltpu.VMEM),
        ],
        out_shape=jax.ShapeDtypeStruct((BATCH, D_PROJ), jnp.float32),
        scratch_shapes=[
            pltpu.VMEM((BATCH, D_IN), jnp.float32),
            pltpu.SemaphoreType.DMA,
        ],
    )(idx_start, table, W, b2)

    return pl.pallas_call(
        _final_body,
        out_shape=jax.ShapeDtypeStruct((BATCH, 2 * D_PROJ), jnp.float32),
    )(gathered_ends, W, b2, start_half)
